# Initial kernel scaffold; baseline (speedup 1.0000x reference)
import jax
import jax.numpy as jnp
from jax.experimental import pallas as pl


def kernel(node_features, adjacency, W_msg, b_msg, W_upd, b_upd, gamma, beta):
    raise NotImplementedError("write your pallas kernel here")


# speedup vs baseline: 4.9917x; 4.9917x over previous
import functools

import jax
import jax.numpy as jnp
from jax import lax
from jax.experimental import pallas as pl
from jax.experimental.pallas import tpu as pltpu
from jax.experimental.pallas import tpu_sc as plsc

N = 10000
D = 128
E = 320000

NC = 2
NS = 16
L = 16
NW = NC * NS
DC = D + L

K = 80
EPT = E // NW
CHUNKS = EPT // K
NPAD = 10240
NPT = NPAD // NS
ZK = 128

ROWS_TC = 2000


def _prep_body(h_ref, wm_ref, bm_ref, wu_ref, bu_ref, a_ref, b_ref, c_ref):
    h = h_ref[...]
    a_ref[...] = jnp.dot(h, wm_ref[0:D, :], preferred_element_type=jnp.float32)
    b_ref[...] = (
        jnp.dot(h, wm_ref[D:, :], preferred_element_type=jnp.float32)
        + bm_ref[...]
    )
    c_ref[...] = (
        jnp.dot(h, wu_ref[0:D, :], preferred_element_type=jnp.float32)
        + bu_ref[...]
    )


def _prep(h, w_msg, b_msg, w_upd, b_upd):
    n = h.shape[0]
    grid = (n // ROWS_TC,)
    row_spec = pl.BlockSpec((ROWS_TC, D), lambda i: (i, 0))
    full_w = pl.BlockSpec((2 * D, D), lambda i: (0, 0))
    full_b = pl.BlockSpec((1, D), lambda i: (0, 0))
    out = jax.ShapeDtypeStruct((n, D), jnp.float32)
    return pl.pallas_call(
        _prep_body,
        grid=grid,
        in_specs=[row_spec, full_w, full_b, full_w, full_b],
        out_specs=[row_spec, row_spec, row_spec],
        out_shape=[out, out, out],
    )(h, w_msg, b_msg.reshape(1, D), w_upd, b_upd.reshape(1, D))


def _sc_body(a_hbm, b_hbm, src_hbm, tgt_hbm, out_msg, cstage,
             acc, idx_s, idx_t, idx_t2, arows, brows, orows,
             cnt_local, cbuf, sem_a, sem_b):
    c = lax.axis_index("c")
    s = lax.axis_index("s")
    wid = s * NC + c
    wid2 = s * NC + (1 - c)
    row0 = s * NPT

    zvec = jnp.zeros((L,), jnp.float32)

    @pl.loop(0, K)
    def _zero_rows(r):
        for j in range(D // L):
            orows[r, pl.ds(j * L, L)] = zvec

    @pl.loop(0, NPAD // L)
    def _zero_cnt(i):
        cnt_local[pl.ds(i * L, L)] = zvec

    for k in range(NPT // K):
        pltpu.sync_copy(orows, acc.at[pl.ds(row0 + k * K, K)])
    plsc.subcore_barrier()

    ones16 = jnp.ones((L,), jnp.float32)

    @pl.loop(0, CHUNKS)
    def _chunk(ch):
        base = wid * EPT + ch * K
        pltpu.sync_copy(src_hbm.at[pl.ds(base, K)], idx_s)
        pltpu.sync_copy(tgt_hbm.at[pl.ds(base, K)], idx_t)
        pltpu.sync_copy(tgt_hbm.at[pl.ds(wid2 * EPT + ch * K, K)], idx_t2)
        cp_a = pltpu.async_copy(a_hbm.at[idx_s], arows, sem_a)
        cp_b = pltpu.async_copy(b_hbm.at[idx_t], brows, sem_b)
        cp_a.wait()
        cp_b.wait()

        @pl.loop(0, K)
        def _rows(r):
            for j in range(D // L):
                va = arows[r, pl.ds(j * L, L)]
                vb = brows[r, pl.ds(j * L, L)]
                orows[r, pl.ds(j * L, L)] = jnp.maximum(va + vb, 0.0)

        pltpu.sync_copy(orows, acc.at[idx_t], add=True)

        for g in range(K // L):
            plsc.addupdate_scatter(cnt_local, [idx_t[pl.ds(g * L, L)]], ones16)
            plsc.addupdate_scatter(cnt_local, [idx_t2[pl.ds(g * L, L)]], ones16)

    for o in range(NS):
        pltpu.sync_copy(cnt_local.at[pl.ds(o * NPT, NPT)], cstage.at[c, o, s])
    plsc.subcore_barrier()

    @pl.loop(0, NPT // L)
    def _zero_sum(j):
        cnt_local[pl.ds(j * L, L)] = zvec

    for t in range(NS):
        pltpu.sync_copy(cstage.at[c, s, t], cbuf)

        @pl.loop(0, NPT // L)
        def _accum(j):
            cnt_local[pl.ds(j * L, L)] = (
                cnt_local[pl.ds(j * L, L)] + cbuf[pl.ds(j * L, L)]
            )

    @pl.loop(0, NPT // L)
    def _recip(j):
        cnt_local[pl.ds(j * L, L)] = 1.0 / jnp.maximum(
            cnt_local[pl.ds(j * L, L)], 1.0
        )

    for k in range(NPT // K):
        r = row0 + k * K
        pltpu.sync_copy(acc.at[pl.ds(r, K)], orows)

        @pl.loop(0, K)
        def _scale(q):
            rec = plsc.load_gather(cnt_local, [jnp.full((L,), k * K + q,
                                                        jnp.int32)])
            for j in range(D // L):
                orows[q, pl.ds(j * L, L)] = orows[q, pl.ds(j * L, L)] * rec

        pltpu.sync_copy(orows, out_msg.at[c, pl.ds(r, K)])


def _sc_scatter(a, b, src, tgt):
    mesh = plsc.VectorSubcoreMesh(core_axis_name="c", subcore_axis_name="s")
    f = pl.kernel(
        _sc_body,
        out_type=(
            jax.ShapeDtypeStruct((NC, NPAD, D), jnp.float32),
            jax.ShapeDtypeStruct((NC, NS, NS, NPT), jnp.float32),
        ),
        mesh=mesh,
        scratch_types=[
            pltpu.VMEM_SHARED((NPAD, D), jnp.float32),
            pltpu.VMEM((K,), jnp.int32),
            pltpu.VMEM((K,), jnp.int32),
            pltpu.VMEM((K,), jnp.int32),
            pltpu.VMEM((K, D), jnp.float32),
            pltpu.VMEM((K, D), jnp.float32),
            pltpu.VMEM((K, D), jnp.float32),
            pltpu.VMEM((NPAD,), jnp.float32),
            pltpu.VMEM((NPT,), jnp.float32),
            pltpu.SemaphoreType.DMA,
            pltpu.SemaphoreType.DMA,
        ],
        compiler_params=pltpu.CompilerParams(needs_layout_passes=False),
    )
    return f(a, b, src, tgt)


def _final_body(h_ref, c_ref, parts_ref, wu_ref, g_ref, be_ref, out_ref):
    messages = parts_ref[0] + parts_ref[1]
    upd = jnp.maximum(
        c_ref[...]
        + jnp.dot(messages, wu_ref[D:, :], preferred_element_type=jnp.float32),
        0.0,
    )
    h2 = upd + h_ref[...]
    mu = jnp.mean(h2, axis=-1, keepdims=True)
    zc = h2 - mu
    var = jnp.mean(zc * zc, axis=-1, keepdims=True)
    out_ref[...] = zc * lax.rsqrt(var + 1e-5) * g_ref[...] + be_ref[...]


def _final(h, c, parts, w_upd, gamma, beta):
    n = h.shape[0]
    grid = (n // ROWS_TC,)
    row_spec = pl.BlockSpec((ROWS_TC, D), lambda i: (i, 0))
    parts_spec = pl.BlockSpec((NC, ROWS_TC, D), lambda i: (0, i, 0))
    full_w = pl.BlockSpec((2 * D, D), lambda i: (0, 0))
    full_b = pl.BlockSpec((1, D), lambda i: (0, 0))
    return pl.pallas_call(
        _final_body,
        grid=grid,
        in_specs=[row_spec, row_spec, parts_spec, full_w, full_b, full_b],
        out_specs=row_spec,
        out_shape=jax.ShapeDtypeStruct((n, D), jnp.float32),
    )(h, c, parts, w_upd, gamma.reshape(1, D), beta.reshape(1, D))


def kernel(node_features, adjacency, W_msg, b_msg, W_upd, b_upd, gamma, beta):
    src = adjacency[:, 0]
    tgt = adjacency[:, 1]
    a, b, c = _prep(node_features, W_msg, b_msg, W_upd, b_upd)
    parts, _ = _sc_scatter(a, b, src, tgt)
    return _final(node_features, c, parts, W_upd, gamma, beta)


# speedup vs baseline: 8.5222x; 1.7073x over previous
import jax
import jax.numpy as jnp
from jax import lax
from jax.experimental import pallas as pl
from jax.experimental.pallas import tpu as pltpu
from jax.experimental.pallas import tpu_sc as plsc

N = 10000
D = 128
E = 320000

NC = 2
NS = 16
L = 16
NW = NC * NS
NPAD = 10240
NPT = NPAD // NS

K = 40
SUP = 10
EPT = E // NW
CHUNKS = EPT // K
NSUP = CHUNKS // SUP

ROWS_TC = 2000


def _prep_body(h_ref, wm_ref, bm_ref, wu_ref, bu_ref, a_ref, b_ref, c_ref):
    h = h_ref[...]
    a_ref[...] = jnp.dot(h, wm_ref[0:D, :], preferred_element_type=jnp.float32)
    b_ref[...] = (
        jnp.dot(h, wm_ref[D:, :], preferred_element_type=jnp.float32)
        + bm_ref[...]
    )
    c_ref[...] = (
        jnp.dot(h, wu_ref[0:D, :], preferred_element_type=jnp.float32)
        + bu_ref[...]
    )


def _prep(h, w_msg, b_msg, w_upd, b_upd):
    n = h.shape[0]
    grid = (n // ROWS_TC,)
    row_spec = pl.BlockSpec((ROWS_TC, D), lambda i: (i, 0))
    full_w = pl.BlockSpec((2 * D, D), lambda i: (0, 0))
    full_b = pl.BlockSpec((1, D), lambda i: (0, 0))
    out = jax.ShapeDtypeStruct((n, D), jnp.float32)
    return pl.pallas_call(
        _prep_body,
        grid=grid,
        in_specs=[row_spec, full_w, full_b, full_w, full_b],
        out_specs=[row_spec, row_spec, row_spec],
        out_shape=[out, out, out],
    )(h, w_msg, b_msg.reshape(1, D), w_upd, b_upd.reshape(1, D))


def _sc_body(a_hbm, b_hbm, src_hbm, tgt_hbm, out_msg, cstage,
             acc,
             isrc0, isrc1, itgt0, itgt1, itg20, itg21,
             stgt0, stgt1,
             ar0, ar1, br0, br1, or0, or1,
             cnt_local, cbuf,
             sem_a, sem_b, sem_o0, sem_o1, sem_i, sem_s):
    c = lax.axis_index("c")
    s = lax.axis_index("s")
    wid = s * NC + c
    wid2 = s * NC + (1 - c)
    row0 = s * NPT
    e0 = wid * EPT
    e02 = wid2 * EPT

    isrc = (isrc0, isrc1)
    itgt = (itgt0, itgt1)
    itg2 = (itg20, itg21)
    stgt = (stgt0, stgt1)
    arows = (ar0, ar1)
    brows = (br0, br1)
    orows = (or0, or1)
    sem_o = (sem_o0, sem_o1)

    zvec = jnp.zeros((L,), jnp.float32)
    ones16 = jnp.ones((L,), jnp.float32)
    tail_mask = lax.broadcasted_iota(jnp.int32, (L,), 0) >= (3 * L - K)

    @pl.loop(0, K)
    def _zero_rows(r):
        for j in range(D // L):
            or0[r, pl.ds(j * L, L)] = zvec

    @pl.loop(0, NPAD // L)
    def _zero_cnt(i):
        cnt_local[pl.ds(i * L, L)] = zvec

    for k in range(NPT // K):
        pltpu.sync_copy(or0, acc.at[pl.ds(row0 + k * K, K)])
    plsc.subcore_barrier()

    def load_super(su, sb):
        base = su * SUP * K
        pltpu.async_copy(src_hbm.at[pl.ds(e0 + base, SUP * K)], isrc[sb],
                         sem_i)
        pltpu.async_copy(tgt_hbm.at[pl.ds(e0 + base, SUP * K)], itgt[sb],
                         sem_i)
        pltpu.async_copy(tgt_hbm.at[pl.ds(e02 + base, SUP * K)], itg2[sb],
                         sem_i)

    def wait_super(sb):
        for buf in (isrc[sb], itgt[sb], itg2[sb]):
            pltpu.make_async_copy(src_hbm.at[pl.ds(0, SUP * K)], buf,
                                  sem_i).wait()

    def issue_gathers(j, sb, p):
        pltpu.async_copy(a_hbm.at[isrc[sb].at[pl.ds(j * K, K)]], arows[p],
                         sem_a)
        pltpu.async_copy(b_hbm.at[itgt[sb].at[pl.ds(j * K, K)]], brows[p],
                         sem_b)

    def wait_gathers(p):
        pltpu.make_async_copy(a_hbm.at[pl.ds(0, K)], arows[p], sem_a).wait()
        pltpu.make_async_copy(b_hbm.at[pl.ds(0, K)], brows[p], sem_b).wait()

    def wait_scatter(p):
        pltpu.make_async_copy(a_hbm.at[pl.ds(0, K)], acc.at[pl.ds(0, K)],
                              sem_o[p]).wait()

    def compute(p):
        @pl.loop(0, K)
        def _rows(r):
            for j in range(D // L):
                va = arows[p][r, pl.ds(j * L, L)]
                vb = brows[p][r, pl.ds(j * L, L)]
                orows[p][r, pl.ds(j * L, L)] = jnp.maximum(va + vb, 0.0)

    def counts(j, sb):
        for ref in (itgt[sb], itg2[sb]):
            for g in range(2):
                plsc.addupdate_scatter(
                    cnt_local, [ref[pl.ds(j * K + g * L, L)]], ones16)
            plsc.addupdate_scatter(
                cnt_local, [ref[pl.ds(j * K + K - L, L)]], ones16,
                mask=tail_mask)

    def load_stgt(cg, p):
        pltpu.async_copy(tgt_hbm.at[pl.ds(e0 + cg * K, K)], stgt[p], sem_s)

    def wait_stgt(p):
        pltpu.make_async_copy(tgt_hbm.at[pl.ds(0, K)], stgt[p], sem_s).wait()

    def issue_scatter(p):
        pltpu.async_copy(orows[p], acc.at[stgt[p]], sem_o[p], add=True)

    def inner(su, sb):
        for j in range(SUP):
            p = j % 2
            cg = su * SUP + j
            if j == 2:
                @pl.when(su + 1 < NSUP)
                def _pref():
                    load_super(su + 1, 1 - sb)
            wait_gathers(p)
            load_stgt(cg, p)
            if j == SUP - 1:
                @pl.when(su + 1 < NSUP)
                def _nxt():
                    wait_super(1 - sb)
                    issue_gathers(0, 1 - sb, 1 - p)
            else:
                issue_gathers(j + 1, sb, 1 - p)
            compute(p)
            counts(j, sb)
            if j == 0:
                @pl.when(su > 0)
                def _ws():
                    wait_scatter(1 - p)
            else:
                wait_scatter(1 - p)
            wait_stgt(p)
            issue_scatter(p)

    load_super(0, 0)
    wait_super(0)
    issue_gathers(0, 0, 0)

    @pl.loop(0, NSUP)
    def _super(su):
        sb = lax.rem(su, 2)

        @pl.when(sb == 0)
        def _even():
            inner(su, 0)

        @pl.when(sb == 1)
        def _odd():
            inner(su, 1)

    wait_scatter(1)

    for o in range(NS):
        pltpu.sync_copy(cnt_local.at[pl.ds(o * NPT, NPT)], cstage.at[c, o, s])
    plsc.subcore_barrier()

    @pl.loop(0, NPT // L)
    def _zero_sum(j):
        cnt_local[pl.ds(j * L, L)] = zvec

    for t in range(NS):
        pltpu.sync_copy(cstage.at[c, s, t], cbuf)

        @pl.loop(0, NPT // L)
        def _accum(j):
            cnt_local[pl.ds(j * L, L)] = (
                cnt_local[pl.ds(j * L, L)] + cbuf[pl.ds(j * L, L)]
            )

    @pl.loop(0, NPT // L)
    def _recip(j):
        cnt_local[pl.ds(j * L, L)] = 1.0 / jnp.maximum(
            cnt_local[pl.ds(j * L, L)], 1.0
        )

    for k in range(NPT // K):
        r = row0 + k * K
        pltpu.sync_copy(acc.at[pl.ds(r, K)], or0)

        @pl.loop(0, K)
        def _scale(q):
            rec = plsc.load_gather(cnt_local, [jnp.full((L,), k * K + q,
                                                        jnp.int32)])
            for j in range(D // L):
                or0[q, pl.ds(j * L, L)] = or0[q, pl.ds(j * L, L)] * rec

        pltpu.sync_copy(or0, out_msg.at[c, pl.ds(r, K)])


def _sc_scatter(a, b, src, tgt):
    mesh = plsc.VectorSubcoreMesh(core_axis_name="c", subcore_axis_name="s")
    f = pl.kernel(
        _sc_body,
        out_type=(
            jax.ShapeDtypeStruct((NC, NPAD, D), jnp.float32),
            jax.ShapeDtypeStruct((NC, NS, NS, NPT), jnp.float32),
        ),
        mesh=mesh,
        scratch_types=[
            pltpu.VMEM_SHARED((NPAD, D), jnp.float32),
            pltpu.VMEM((SUP * K,), jnp.int32),
            pltpu.VMEM((SUP * K,), jnp.int32),
            pltpu.VMEM((SUP * K,), jnp.int32),
            pltpu.VMEM((SUP * K,), jnp.int32),
            pltpu.VMEM((SUP * K,), jnp.int32),
            pltpu.VMEM((SUP * K,), jnp.int32),
            pltpu.VMEM((K,), jnp.int32),
            pltpu.VMEM((K,), jnp.int32),
            pltpu.VMEM((K, D), jnp.float32),
            pltpu.VMEM((K, D), jnp.float32),
            pltpu.VMEM((K, D), jnp.float32),
            pltpu.VMEM((K, D), jnp.float32),
            pltpu.VMEM((K, D), jnp.float32),
            pltpu.VMEM((K, D), jnp.float32),
            pltpu.VMEM((NPAD,), jnp.float32),
            pltpu.VMEM((NPT,), jnp.float32),
            pltpu.SemaphoreType.DMA,
            pltpu.SemaphoreType.DMA,
            pltpu.SemaphoreType.DMA,
            pltpu.SemaphoreType.DMA,
            pltpu.SemaphoreType.DMA,
            pltpu.SemaphoreType.DMA,
        ],
        compiler_params=pltpu.CompilerParams(needs_layout_passes=False),
    )
    return f(a, b, src, tgt)


def _final_body(h_ref, c_ref, parts_ref, wu_ref, g_ref, be_ref, out_ref):
    messages = parts_ref[0] + parts_ref[1]
    upd = jnp.maximum(
        c_ref[...]
        + jnp.dot(messages, wu_ref[D:, :], preferred_element_type=jnp.float32),
        0.0,
    )
    h2 = upd + h_ref[...]
    mu = jnp.mean(h2, axis=-1, keepdims=True)
    zc = h2 - mu
    var = jnp.mean(zc * zc, axis=-1, keepdims=True)
    out_ref[...] = zc * lax.rsqrt(var + 1e-5) * g_ref[...] + be_ref[...]


def _final(h, c, parts, w_upd, gamma, beta):
    n = h.shape[0]
    grid = (n // ROWS_TC,)
    row_spec = pl.BlockSpec((ROWS_TC, D), lambda i: (i, 0))
    parts_spec = pl.BlockSpec((NC, ROWS_TC, D), lambda i: (0, i, 0))
    full_w = pl.BlockSpec((2 * D, D), lambda i: (0, 0))
    full_b = pl.BlockSpec((1, D), lambda i: (0, 0))
    return pl.pallas_call(
        _final_body,
        grid=grid,
        in_specs=[row_spec, row_spec, parts_spec, full_w, full_b, full_b],
        out_specs=row_spec,
        out_shape=jax.ShapeDtypeStruct((n, D), jnp.float32),
    )(h, c, parts, w_upd, gamma.reshape(1, D), beta.reshape(1, D))


def kernel(node_features, adjacency, W_msg, b_msg, W_upd, b_upd, gamma, beta):
    src = adjacency[:, 0]
    tgt = adjacency[:, 1]
    a, b, c = _prep(node_features, W_msg, b_msg, W_upd, b_upd)
    parts, _ = _sc_scatter(a, b, src, tgt)
    return _final(node_features, c, parts, W_upd, gamma, beta)
